# Initial kernel scaffold; baseline (speedup 1.0000x reference)
import jax
import jax.numpy as jnp
from jax.experimental import pallas as pl


def kernel(inputs0, logits0, logits1, targets, indexes, neighbors, neighbor_dists, rampup, features):
    raise NotImplementedError("write your pallas kernel here")


# speedup vs baseline: 38.9596x; 38.9596x over previous
import functools

import jax
import jax.numpy as jnp
from jax import lax
from jax.experimental import pallas as pl
from jax.experimental.pallas import tpu as pltpu

_B = 512
_C = 8192
_F = 2048
_K = 20
_TEMP = 0.05
_TEMP_DIST = 0.05
_ALPHA = 0.9

_NCE_CW = 512
_NCE_NC = _C // _NCE_CW
_ROW_BLK = 128
_MAIN_CW = 512
_MAIN_NC = _C // _MAIN_CW


def _nce_body(x_ref, t_ref, feat_ref, out_ref, xs, ms, ss, ts):
    i = pl.program_id(0)

    @pl.when(i == 0)
    def _init():
        x = x_ref[...]
        inv = lax.rsqrt(jnp.sum(x * x, axis=1, keepdims=True))
        xs[...] = x * inv
        ms[...] = jnp.full((_B, 1), -1e30, jnp.float32)
        ss[...] = jnp.zeros((_B, 1), jnp.float32)
        ts[...] = jnp.zeros((_B, 1), jnp.float32)

    blk = lax.dot_general(
        xs[...], feat_ref[...], (((1,), (1,)), ((), ())),
        preferred_element_type=jnp.float32) * (1.0 / _TEMP)
    m_old = ms[...]
    m_new = jnp.maximum(m_old, jnp.max(blk, axis=1, keepdims=True))
    ss[...] = (ss[...] * jnp.exp(m_old - m_new)
               + jnp.sum(jnp.exp(blk - m_new), axis=1, keepdims=True))
    ms[...] = m_new
    col = i * _NCE_CW + lax.broadcasted_iota(jnp.int32, (_B, _NCE_CW), 1)
    ts[...] += jnp.sum(jnp.where(col == t_ref[...], blk, 0.0), axis=1,
                       keepdims=True)

    @pl.when(i == _NCE_NC - 1)
    def _fin():
        lse = ms[...] + jnp.log(ss[...])
        out_ref[...] = (jnp.sum(lse - ts[...]) * (1.0 / _B)).reshape(1, 1)


def _stats_body(l0_ref, l1_ref, t_ref, lse0_ref, lse1_ref, ce_ref):
    i = pl.program_id(0)
    l0 = l0_ref[...]
    m0 = jnp.max(l0, axis=1, keepdims=True)
    lse0 = m0 + jnp.log(jnp.sum(jnp.exp(l0 - m0), axis=1, keepdims=True))
    lse0_ref[...] = lse0
    l1 = l1_ref[...]
    m1 = jnp.max(l1, axis=1, keepdims=True)
    lse1_ref[...] = m1 + jnp.log(
        jnp.sum(jnp.exp(l1 - m1), axis=1, keepdims=True))
    col = lax.broadcasted_iota(jnp.int32, (_ROW_BLK, _C), 1)
    tgt = jnp.sum(jnp.where(col == t_ref[...], l0, 0.0), axis=1, keepdims=True)

    @pl.when(i == 0)
    def _init():
        ce_ref[...] = jnp.zeros((1, 1), jnp.float32)

    ce_ref[...] += jnp.sum(tgt - lse0)


def _am_body(nb_ref, d_ref, a_ref, m_ref):
    d = d_ref[...]
    e = jnp.exp(d * (1.0 / _TEMP_DIST))
    wk = e / (2.0 * jnp.sum(e, axis=1, keepdims=True))
    nb = nb_ref[...]
    col = lax.broadcasted_iota(jnp.int32, (_B, _B), 1)
    a = jnp.zeros((_B, _B), jnp.float32)
    m = jnp.zeros((_B, _B), jnp.float32)
    for k in range(_K):
        hit = col == nb[:, k:k + 1]
        a = a + jnp.where(hit, wk[:, k:k + 1], 0.0)
        m = m + jnp.where(hit, 1.0 / _K, 0.0)
    a_ref[...] = a
    m_ref[...] = m


def _main_body(l0_ref, l1_ref, lse0_ref, lse1_ref, a_ref, m_ref,
               acc1_ref, acc2_ref, acc3_ref):
    i = pl.program_id(0)
    lsm0 = l0_ref[...] - lse0_ref[...]
    p0 = jnp.exp(lsm0)
    lsm1 = l1_ref[...] - lse1_ref[...]
    p1 = jnp.exp(lsm1)
    t = lax.dot_general(m_ref[...], p0, (((1,), (0,)), ((), ())),
                        preferred_element_type=jnp.float32)
    s = lax.dot_general(a_ref[...], p0 + p1, (((1,), (0,)), ((), ())),
                        preferred_element_type=jnp.float32)
    tlogt = jnp.where(t > 0.0, t * jnp.log(jnp.where(t > 0.0, t, 1.0)), 0.0)

    @pl.when(i == 0)
    def _init():
        acc1_ref[...] = jnp.zeros((1, 1), jnp.float32)
        acc2_ref[...] = jnp.zeros((1, 1), jnp.float32)
        acc3_ref[...] = jnp.zeros((1, 1), jnp.float32)

    acc1_ref[...] += jnp.sum(tlogt)
    acc2_ref[...] += jnp.sum(t * lsm1)
    acc3_ref[...] += jnp.sum(s * lsm0)


def kernel(inputs0, logits0, logits1, targets, indexes, neighbors,
           neighbor_dists, rampup, features):
    del indexes
    t2d = targets.reshape(_B, 1)

    loss_nce = pl.pallas_call(
        _nce_body,
        grid=(_NCE_NC,),
        in_specs=[
            pl.BlockSpec((_B, _F), lambda i: (0, 0)),
            pl.BlockSpec((_B, 1), lambda i: (0, 0)),
            pl.BlockSpec((_NCE_CW, _F), lambda i: (i, 0)),
        ],
        out_specs=pl.BlockSpec((1, 1), lambda i: (0, 0)),
        out_shape=jax.ShapeDtypeStruct((1, 1), jnp.float32),
        scratch_shapes=[
            pltpu.VMEM((_B, _F), jnp.float32),
            pltpu.VMEM((_B, 1), jnp.float32),
            pltpu.VMEM((_B, 1), jnp.float32),
            pltpu.VMEM((_B, 1), jnp.float32),
        ],
    )(inputs0, t2d, features)

    lse0, lse1, ce_tgt = pl.pallas_call(
        _stats_body,
        grid=(_B // _ROW_BLK,),
        in_specs=[
            pl.BlockSpec((_ROW_BLK, _C), lambda i: (i, 0)),
            pl.BlockSpec((_ROW_BLK, _C), lambda i: (i, 0)),
            pl.BlockSpec((_ROW_BLK, 1), lambda i: (i, 0)),
        ],
        out_specs=[
            pl.BlockSpec((_ROW_BLK, 1), lambda i: (i, 0)),
            pl.BlockSpec((_ROW_BLK, 1), lambda i: (i, 0)),
            pl.BlockSpec((1, 1), lambda i: (0, 0)),
        ],
        out_shape=[
            jax.ShapeDtypeStruct((_B, 1), jnp.float32),
            jax.ShapeDtypeStruct((_B, 1), jnp.float32),
            jax.ShapeDtypeStruct((1, 1), jnp.float32),
        ],
    )(logits0, logits1, t2d)

    a_mat, m_mat = pl.pallas_call(
        _am_body,
        in_specs=[
            pl.BlockSpec((_B, _K), lambda: (0, 0)),
            pl.BlockSpec((_B, _K), lambda: (0, 0)),
        ],
        out_specs=[
            pl.BlockSpec((_B, _B), lambda: (0, 0)),
            pl.BlockSpec((_B, _B), lambda: (0, 0)),
        ],
        out_shape=[
            jax.ShapeDtypeStruct((_B, _B), jnp.float32),
            jax.ShapeDtypeStruct((_B, _B), jnp.float32),
        ],
    )(neighbors, neighbor_dists)

    acc1, acc2, acc3 = pl.pallas_call(
        _main_body,
        grid=(_MAIN_NC,),
        in_specs=[
            pl.BlockSpec((_B, _MAIN_CW), lambda i: (0, i)),
            pl.BlockSpec((_B, _MAIN_CW), lambda i: (0, i)),
            pl.BlockSpec((_B, 1), lambda i: (0, 0)),
            pl.BlockSpec((_B, 1), lambda i: (0, 0)),
            pl.BlockSpec((_B, _B), lambda i: (0, 0)),
            pl.BlockSpec((_B, _B), lambda i: (0, 0)),
        ],
        out_specs=[
            pl.BlockSpec((1, 1), lambda i: (0, 0)),
            pl.BlockSpec((1, 1), lambda i: (0, 0)),
            pl.BlockSpec((1, 1), lambda i: (0, 0)),
        ],
        out_shape=[
            jax.ShapeDtypeStruct((1, 1), jnp.float32),
            jax.ShapeDtypeStruct((1, 1), jnp.float32),
            jax.ShapeDtypeStruct((1, 1), jnp.float32),
        ],
    )(logits0, logits1, lse0, lse1, a_mat, m_mat)

    loss_ce = -(_ALPHA * ce_tgt[0, 0] + (1.0 - _ALPHA) * acc3[0, 0]) / _B
    loss_kl = (acc1[0, 0] - acc2[0, 0]) / _B
    return (loss_nce[0, 0], loss_ce, rampup * loss_kl)
